# baseline (speedup 1.0000x reference)
import functools

import jax
import jax.numpy as jnp
from jax import lax
from jax.experimental import pallas as pl
from jax.experimental.pallas import tpu as pltpu
from jax.experimental.pallas import tpu_sc as plsc

VOCAB = 1000000
EMBED_DIM = 64
BATCH = 4096
HIST_LEN = 200

NC = 2
NS = 16
NW = NC * NS
L = 16

NPAIR = VOCAB // 2


def _gather_body(idx_hbm, r_hbm, out_hbm, raw_v, pidx_v, g_v, b_v, gsems, ssems):
    w = lax.axis_index("s") * NC + lax.axis_index("c")
    iota = lax.iota(jnp.int32, L)

    def load_idx(t, buf):
        pltpu.sync_copy(idx_hbm.at[t, pl.ds(128 * w, 128)], raw_v.at[buf])
        for m in range(8):
            v = raw_v[buf, pl.ds(16 * m, 16)]
            pidx_v[buf, pl.ds(16 * m, 16)] = lax.shift_right_logical(v, 1)

    def start_gather(buf):
        pltpu.async_copy(r_hbm.at[pidx_v.at[buf]], g_v.at[buf], gsems.at[buf])

    def wait_gather(buf):
        pltpu.make_async_copy(r_hbm.at[pidx_v.at[buf]], g_v.at[buf],
                              gsems.at[buf]).wait()

    def transpose(buf):
        def row(d, _):
            for m in range(8):
                l2 = iota + 16 * m
                par = lax.bitwise_and(raw_v[buf, pl.ds(16 * m, 16)], 1)
                col = par * 64 + d
                b_v[buf, d, pl.ds(16 * m, 16)] = plsc.load_gather(
                    g_v.at[buf], [l2, col])
            return ()
        lax.fori_loop(0, EMBED_DIM, row, (), unroll=False)

    def start_store(t, buf):
        pltpu.async_copy(b_v.at[buf],
                         out_hbm.at[t, pl.ds(0, EMBED_DIM), pl.ds(128 * w, 128)],
                         ssems.at[buf])

    def wait_store(t, buf):
        pltpu.make_async_copy(b_v.at[buf],
                              out_hbm.at[t, pl.ds(0, EMBED_DIM),
                                         pl.ds(128 * w, 128)],
                              ssems.at[buf]).wait()

    load_idx(0, 0)
    start_gather(0)

    def step(t, _):
        b = lax.rem(t, 2)
        nb = lax.rem(t + 1, 2)

        @pl.when(t + 1 < HIST_LEN)
        def _():
            load_idx(t + 1, nb)
            start_gather(nb)

        wait_gather(b)

        @pl.when(t >= 2)
        def _():
            wait_store(t - 2, b)
        transpose(b)
        start_store(t, b)
        return ()

    lax.fori_loop(0, HIST_LEN, step, (), unroll=False)
    wait_store(HIST_LEN - 2, 0)
    wait_store(HIST_LEN - 1, 1)


@jax.jit
def _gather(idx_t, r_tbl):
    mesh = plsc.VectorSubcoreMesh(core_axis_name="c", subcore_axis_name="s")
    kern = pl.kernel(
        _gather_body,
        out_type=jax.ShapeDtypeStruct((HIST_LEN, EMBED_DIM, BATCH), jnp.float32),
        mesh=mesh,
        scratch_types=[
            pltpu.VMEM((2, 128), jnp.int32),
            pltpu.VMEM((2, 128), jnp.int32),
            pltpu.VMEM((2, 128, 128), jnp.float32),
            pltpu.VMEM((2, EMBED_DIM, 128), jnp.float32),
            pltpu.SemaphoreType.DMA((2,)),
            pltpu.SemaphoreType.DMA((2,)),
        ],
        compiler_params=pltpu.CompilerParams(
            use_tc_tiling_on_sc=True, needs_layout_passes=False),
    )
    return kern(idx_t, r_tbl)


def kernel(inputs, embeddings):
    idx_t = inputs.astype(jnp.int32).T
    r_tbl = embeddings.reshape(NPAIR, 128)
    out_t = _gather(idx_t, r_tbl)
    return out_t.transpose(2, 0, 1)


# speedup vs baseline: 1.7658x; 1.7658x over previous
import jax
import jax.numpy as jnp
from jax import lax
from jax.experimental import pallas as pl
from jax.experimental.pallas import tpu as pltpu
from jax.experimental.pallas import tpu_sc as plsc

VOCAB = 1000000
EMBED_DIM = 64
BATCH = 4096
HIST_LEN = 200

NC = 2
NS = 16
NW = NC * NS
L = 16

NPAIR = VOCAB // 2
NBLK = BATCH * HIST_LEN // 128
BPW = NBLK // NW
NBUF = 4
SBUF = 2


def _gather_body(idx_hbm, tbl_hbm, out_hbm, slab_v, ibuf_v, pofs_v, g_v,
                 sel_v, gsems, ssems):
    w = lax.axis_index("s") * NC + lax.axis_index("c")
    iota = lax.iota(jnp.int32, L)

    pltpu.sync_copy(idx_hbm.at[pl.ds(BPW * w, BPW)], slab_v)

    def build(blk, b):
        for m in range(8):
            v = slab_v[blk, pl.ds(L * m, L)]
            ibuf_v[b, pl.ds(L * m, L)] = lax.shift_right_logical(v, 1)
            pofs_v[b, pl.ds(L * m, L)] = lax.shift_left(
                lax.bitwise_and(v, 1), 6)

    def fire(blk, b):
        build(blk, b)
        pltpu.async_copy(tbl_hbm.at[ibuf_v.at[b]], g_v.at[b], gsems.at[b])

    def gwait(b):
        pltpu.make_async_copy(tbl_hbm.at[ibuf_v.at[b]], g_v.at[b],
                              gsems.at[b]).wait()

    def select(b, s):
        def rowpair(a, _):
            for d in range(2):
                k = 2 * a + d
                pm = plsc.load_gather(pofs_v.at[b],
                                      [lax.broadcast(k, (L,))]) + iota
                for c in range(4):
                    sel_v[s, a, pl.ds(64 * d + L * c, L)] = plsc.load_gather(
                        g_v.at[b, k], [pm + L * c])
            return ()
        lax.fori_loop(0, 64, rowpair, (), unroll=2)

    def store(blk, s):
        pltpu.async_copy(sel_v.at[s], out_hbm.at[BPW * w + blk],
                         ssems.at[s])

    def swait(blk, s):
        pltpu.make_async_copy(sel_v.at[s], out_hbm.at[BPW * w + blk],
                              ssems.at[s]).wait()

    def step(blk, b, s, prev_store, next_fire):
        gwait(b)
        if prev_store:
            swait(blk - SBUF, s)
        select(b, s)
        store(blk, s)
        if next_fire:
            fire(blk + NBUF, b)

    for b in range(NBUF):
        fire(b, b)
    for blk in range(NBUF):
        step(blk, blk, blk % SBUF, blk >= SBUF, True)

    def body(gi, _):
        for b in range(NBUF):
            blk = NBUF * gi + b
            step(blk, b, b % SBUF, True, True)
        return ()

    lax.fori_loop(1, BPW // NBUF - 1, body, (), unroll=False)

    for b in range(NBUF):
        blk = BPW - NBUF + b
        step(blk, b, b % SBUF, True, False)
    for s in range(SBUF):
        swait(BPW - SBUF + s, s)


@jax.jit
def _gather(idx, tbl):
    mesh = plsc.VectorSubcoreMesh(core_axis_name="c", subcore_axis_name="s")
    kern = pl.kernel(
        _gather_body,
        out_type=jax.ShapeDtypeStruct((NBLK, 64, 128), jnp.float32),
        mesh=mesh,
        scratch_types=[
            pltpu.VMEM((BPW, 128), jnp.int32),
            pltpu.VMEM((NBUF, 128), jnp.int32),
            pltpu.VMEM((NBUF, 128), jnp.int32),
            pltpu.VMEM((NBUF, 128, 128), jnp.float32),
            pltpu.VMEM((SBUF, 64, 128), jnp.float32),
            pltpu.SemaphoreType.DMA((NBUF,)),
            pltpu.SemaphoreType.DMA((SBUF,)),
        ],
        compiler_params=pltpu.CompilerParams(
            use_tc_tiling_on_sc=True, needs_layout_passes=False),
    )
    return kern(idx, tbl)


def kernel(inputs, embeddings):
    idx = inputs.astype(jnp.int32).reshape(NBLK, 128)
    tbl = embeddings.reshape(NPAIR, 128)
    out = _gather(idx, tbl)
    return out.reshape(BATCH, HIST_LEN, EMBED_DIM)
